# Initial kernel scaffold; baseline (speedup 1.0000x reference)
import functools

import jax
import jax.numpy as jnp
from jax import lax
from jax.experimental import pallas as pl
from jax.experimental.pallas import tpu as pltpu
from jax.experimental.pallas import tpu_sc as plsc

BATCH = 16384
HIST = 50
D = 64

NC, NS = 2, 16
NW = NC * NS
B_TOT = BATCH * HIST
B_PER_W = B_TOT // NW
IDXW = 128
K = 4
C = IDXW * K
G = B_PER_W // C
ROWS_PER_CHUNK = C // IDXW


def _emb_body(ids_hbm, table_hbm, out_hbm, idx_v, rows_v, gat_sem):
    wid = lax.axis_index("s") * NC + lax.axis_index("c")
    idx_row_base = wid * (B_PER_W // IDXW)
    out_base = wid * B_PER_W

    def body(g, carry):
        idx_row = idx_row_base + g * ROWS_PER_CHUNK
        off = out_base + g * C
        pltpu.sync_copy(ids_hbm.at[pl.ds(idx_row, ROWS_PER_CHUNK)], idx_v)
        descs = []
        for j in range(K):
            descs.append(
                pltpu.async_copy(
                    table_hbm.at[idx_v.at[j]],
                    rows_v.at[pl.ds(j * IDXW, IDXW)],
                    gat_sem,
                )
            )
        for d in descs:
            d.wait()
        pltpu.sync_copy(rows_v, out_hbm.at[pl.ds(off, C)])
        return carry

    lax.fori_loop(0, G, body, 0)


_mesh = plsc.VectorSubcoreMesh(
    core_axis_name="c", subcore_axis_name="s", num_cores=NC, num_subcores=NS
)

_emb = functools.partial(
    pl.kernel,
    out_type=jax.ShapeDtypeStruct((B_TOT, D), jnp.float32),
    mesh=_mesh,
    scratch_types=[
        pltpu.VMEM((ROWS_PER_CHUNK, IDXW), jnp.int32),
        pltpu.VMEM((C, D), jnp.float32),
        pltpu.SemaphoreType.DMA,
    ],
)(_emb_body)


@jax.jit
def _run(ids2, table):
    return _emb(ids2, table)


def kernel(entity_ids, table):
    ids2 = entity_ids.reshape(B_TOT // IDXW, IDXW).astype(jnp.int32)
    out = _run(ids2, table)
    return out.reshape(BATCH, HIST, D)


# speedup vs baseline: 1.7975x; 1.7975x over previous
import functools

import jax
import jax.numpy as jnp
from jax import lax
from jax.experimental import pallas as pl
from jax.experimental.pallas import tpu as pltpu
from jax.experimental.pallas import tpu_sc as plsc

BATCH = 16384
HIST = 50
D = 64

NC, NS = 2, 16
NW = NC * NS
B_TOT = BATCH * HIST
B_PER_W = B_TOT // NW
IDXW = 128
K = 4
C = IDXW * K
G = B_PER_W // C
ROWS_PER_CHUNK = C // IDXW


def _emb_body(ids_hbm, table_hbm, out_hbm, idx_v, rows_v, gat_sem):
    wid = lax.axis_index("s") * NC + lax.axis_index("c")
    idx_row_base = wid * (B_PER_W // IDXW)
    out_base = wid * B_PER_W

    def body(g, carry):
        idx_row = idx_row_base + g * ROWS_PER_CHUNK
        off = out_base + g * C
        pltpu.sync_copy(ids_hbm.at[pl.ds(idx_row, ROWS_PER_CHUNK)], idx_v)
        descs = []
        for j in range(K):
            descs.append(
                pltpu.async_copy(
                    table_hbm.at[idx_v.at[j]],
                    rows_v.at[pl.ds(j * IDXW, IDXW)],
                    gat_sem,
                )
            )
        for d in descs:
            d.wait()
        pltpu.sync_copy(rows_v, out_hbm.at[pl.ds(off, C)])
        return carry

    lax.fori_loop(0, G, body, 0)


_mesh = plsc.VectorSubcoreMesh(
    core_axis_name="c", subcore_axis_name="s", num_cores=NC, num_subcores=NS
)

_emb = functools.partial(
    pl.kernel,
    out_type=jax.ShapeDtypeStruct((B_TOT, D), jnp.float32),
    mesh=_mesh,
    scratch_types=[
        pltpu.VMEM((ROWS_PER_CHUNK, IDXW), jnp.int32),
        pltpu.VMEM((C, D), jnp.float32),
        pltpu.SemaphoreType.DMA,
    ],
    compiler_params=pltpu.CompilerParams(use_tc_tiling_on_sc=False),
)(_emb_body)


@jax.jit
def _run(ids2, table):
    return _emb(ids2, table)


def kernel(entity_ids, table):
    ids2 = entity_ids.reshape(B_TOT // IDXW, IDXW).astype(jnp.int32)
    out = _run(ids2, table)
    return out.reshape(BATCH, HIST, D)


# speedup vs baseline: 1.8716x; 1.0412x over previous
import functools

import jax
import jax.numpy as jnp
from jax import lax
from jax.experimental import pallas as pl
from jax.experimental.pallas import tpu as pltpu
from jax.experimental.pallas import tpu_sc as plsc

BATCH = 16384
HIST = 50
D = 64

NC, NS = 2, 16
NW = NC * NS
B_TOT = BATCH * HIST
B_PER_W = B_TOT // NW
IDXW = 128
K = 4
C = IDXW * K
G = B_PER_W // C
ROWS_PER_CHUNK = C // IDXW
NBUF = 2


def _emb_body(ids_hbm, table_hbm, out_hbm, idx_v, rows_v, gat_sem, out_sem):
    wid = lax.axis_index("s") * NC + lax.axis_index("c")
    idx_row_base = wid * (B_PER_W // IDXW)
    out_base = wid * B_PER_W

    pltpu.sync_copy(ids_hbm.at[pl.ds(idx_row_base, B_PER_W // IDXW)], idx_v)

    def do_gathers(g, b):
        descs = []
        for j in range(K):
            descs.append(
                pltpu.async_copy(
                    table_hbm.at[idx_v.at[g * ROWS_PER_CHUNK + j]],
                    rows_v.at[b].at[pl.ds(j * IDXW, IDXW)],
                    gat_sem,
                )
            )
        return descs

    def start_store(g, b):
        pltpu.async_copy(
            rows_v.at[b], out_hbm.at[pl.ds(out_base + g * C, C)], out_sem
        )

    def drain_one_store(b):
        pltpu.make_async_copy(
            rows_v.at[b], out_hbm.at[pl.ds(out_base, C)], out_sem
        ).wait()

    def outer(gg, carry):
        for b in range(NBUF):
            g = gg * NBUF + b

            @pl.when(gg > 0)
            def _():
                drain_one_store(b)

            descs = do_gathers(g, b)
            for d in descs:
                d.wait()
            start_store(g, b)
        return carry

    lax.fori_loop(0, G // NBUF, outer, 0)

    for b in range(NBUF):
        drain_one_store(b)


_mesh = plsc.VectorSubcoreMesh(
    core_axis_name="c", subcore_axis_name="s", num_cores=NC, num_subcores=NS
)

_emb = functools.partial(
    pl.kernel,
    out_type=jax.ShapeDtypeStruct((B_TOT, D), jnp.float32),
    mesh=_mesh,
    scratch_types=[
        pltpu.VMEM((B_PER_W // IDXW, IDXW), jnp.int32),
        pltpu.VMEM((NBUF, C, D), jnp.float32),
        pltpu.SemaphoreType.DMA,
        pltpu.SemaphoreType.DMA,
    ],
    compiler_params=pltpu.CompilerParams(use_tc_tiling_on_sc=False),
)(_emb_body)


@jax.jit
def _run(ids2, table):
    return _emb(ids2, table)


def kernel(entity_ids, table):
    ids2 = entity_ids.reshape(B_TOT // IDXW, IDXW).astype(jnp.int32)
    out = _run(ids2, table)
    return out.reshape(BATCH, HIST, D)


# speedup vs baseline: 1.8754x; 1.0021x over previous
import functools

import jax
import jax.numpy as jnp
from jax import lax
from jax.experimental import pallas as pl
from jax.experimental.pallas import tpu as pltpu
from jax.experimental.pallas import tpu_sc as plsc

BATCH = 16384
HIST = 50
D = 64

NC, NS = 2, 16
NW = NC * NS
B_TOT = BATCH * HIST
B_PER_W = B_TOT // NW
IDXW = 128
K = 2
C = IDXW * K
G = B_PER_W // C
ROWS_PER_CHUNK = C // IDXW
NBUF = 5
LEAD = 3


def _emb_body(ids_hbm, table_hbm, out_hbm, idx_v, rows_v, *sems):
    gat_sems = sems[:NBUF]
    out_sem = sems[NBUF]
    wid = lax.axis_index("s") * NC + lax.axis_index("c")
    idx_row_base = wid * (B_PER_W // IDXW)
    out_base = wid * B_PER_W

    pltpu.sync_copy(ids_hbm.at[pl.ds(idx_row_base, B_PER_W // IDXW)], idx_v)

    def issue_gathers(g, b):
        for j in range(K):
            pltpu.async_copy(
                table_hbm.at[idx_v.at[g * ROWS_PER_CHUNK + j]],
                rows_v.at[b].at[pl.ds(j * IDXW, IDXW)],
                gat_sems[b],
            )

    def wait_gathers(b):
        for j in range(K):
            pltpu.make_async_copy(
                table_hbm.at[pl.ds(0, IDXW)],
                rows_v.at[b].at[pl.ds(j * IDXW, IDXW)],
                gat_sems[b],
            ).wait()

    def start_store(g, b):
        pltpu.async_copy(
            rows_v.at[b], out_hbm.at[pl.ds(out_base + g * C, C)], out_sem
        )

    def drain_one_store():
        pltpu.make_async_copy(
            rows_v.at[0], out_hbm.at[pl.ds(out_base, C)], out_sem
        ).wait()

    for g0 in range(LEAD):
        issue_gathers(g0, g0 % NBUF)

    def outer(gg, carry):
        for b in range(NBUF):
            g = gg * NBUF + b

            @pl.when(g >= NBUF - LEAD)
            def _():
                drain_one_store()

            @pl.when(g < G - LEAD)
            def _():
                issue_gathers(g + LEAD, (b + LEAD) % NBUF)

            wait_gathers(b)
            start_store(g, b)
        return carry

    lax.fori_loop(0, G // NBUF, outer, 0)

    for _ in range(NBUF - LEAD):
        drain_one_store()


_mesh = plsc.VectorSubcoreMesh(
    core_axis_name="c", subcore_axis_name="s", num_cores=NC, num_subcores=NS
)

_emb = functools.partial(
    pl.kernel,
    out_type=jax.ShapeDtypeStruct((B_TOT, D), jnp.float32),
    mesh=_mesh,
    scratch_types=[
        pltpu.VMEM((B_PER_W // IDXW, IDXW), jnp.int32),
        pltpu.VMEM((NBUF, C, D), jnp.float32),
    ]
    + [pltpu.SemaphoreType.DMA] * (NBUF + 1),
    compiler_params=pltpu.CompilerParams(use_tc_tiling_on_sc=False),
)(_emb_body)


@jax.jit
def _run(ids2, table):
    return _emb(ids2, table)


def kernel(entity_ids, table):
    ids2 = entity_ids.reshape(B_TOT // IDXW, IDXW).astype(jnp.int32)
    out = _run(ids2, table)
    return out.reshape(BATCH, HIST, D)


# speedup vs baseline: 1.8787x; 1.0017x over previous
import functools

import jax
import jax.numpy as jnp
from jax import lax
from jax.experimental import pallas as pl
from jax.experimental.pallas import tpu as pltpu
from jax.experimental.pallas import tpu_sc as plsc

BATCH = 16384
HIST = 50
D = 64

NC, NS = 2, 16
NW = NC * NS
BW = BATCH // NW
NBG = BW // 128
G = HIST * NBG
NBUF = 8
K = 4


def _emb_body(ids_hbm, table_hbm, out_hbm, ids_v, rows_v, *sems):
    gat_sems = sems[:NBUF]
    st_sems = sems[NBUF:]
    wid = lax.axis_index("s") * NC + lax.axis_index("c")

    pltpu.sync_copy(ids_hbm.at[:, pl.ds(wid * NBG, NBG), :], ids_v)

    def issue_gather(g, s):
        h = g // NBG
        j = g % NBG
        pltpu.async_copy(
            table_hbm.at[ids_v.at[h, j]], rows_v.at[s], gat_sems[s]
        )

    def wait_gather(s):
        pltpu.make_async_copy(
            table_hbm.at[pl.ds(0, 128)], rows_v.at[s], gat_sems[s]
        ).wait()

    def issue_store(g, s):
        h = g // NBG
        j = g % NBG
        bb = wid * NBG + j
        pltpu.async_copy(
            rows_v.at[s], out_hbm.at[pl.ds(bb * 128, 128), h, :], st_sems[s]
        )

    def wait_store(s):
        pltpu.make_async_copy(
            rows_v.at[s], out_hbm.at[pl.ds(0, 128), 0, :], st_sems[s]
        ).wait()

    for s in range(K):
        issue_gather(s, s)

    def outer(gg, carry):
        for i in range(NBUF):
            g = gg * NBUF + i
            gp = g + K
            s2 = (i + K) % NBUF

            @pl.when(jnp.logical_and(gp < G, gp >= NBUF))
            def _():
                wait_store(s2)

            @pl.when(gp < G)
            def _():
                issue_gather(gp, s2)

            wait_gather(i)
            issue_store(g, i)
        return carry

    lax.fori_loop(0, G // NBUF, outer, 0)

    for s in range(NBUF):
        wait_store(s)


_mesh = plsc.VectorSubcoreMesh(
    core_axis_name="c", subcore_axis_name="s", num_cores=NC, num_subcores=NS
)

_emb = functools.partial(
    pl.kernel,
    out_type=jax.ShapeDtypeStruct((BATCH, HIST, D), jnp.float32),
    mesh=_mesh,
    scratch_types=[
        pltpu.VMEM((HIST, NBG, 128), jnp.int32),
        pltpu.VMEM((NBUF, 128, D), jnp.float32),
    ]
    + [pltpu.SemaphoreType.DMA] * (2 * NBUF),
    compiler_params=pltpu.CompilerParams(use_tc_tiling_on_sc=False),
)(_emb_body)


def kernel(entity_ids, table):
    ids3 = entity_ids.astype(jnp.int32).T.reshape(HIST, BATCH // 128, 128)
    return _emb(ids3, table)


# speedup vs baseline: 1.8792x; 1.0003x over previous
import functools

import jax
import jax.numpy as jnp
from jax import lax
from jax.experimental import pallas as pl
from jax.experimental.pallas import tpu as pltpu
from jax.experimental.pallas import tpu_sc as plsc

BATCH = 16384
HIST = 50
D = 64

NC, NS = 2, 16
NW = NC * NS
BW = BATCH // NW
NBG = BW // 128
G = HIST * NBG
NBUF = 10
K = 5


def _emb_body(ids_hbm, table_hbm, out_hbm, ids_v, rows_v, *sems):
    gat_sems = sems[:NBUF]
    st_sems = sems[NBUF:]
    wid = lax.axis_index("s") * NC + lax.axis_index("c")

    pltpu.sync_copy(ids_hbm.at[:, pl.ds(wid * NBG, NBG), :], ids_v)

    def issue_gather(g, s):
        h = g // NBG
        j = g % NBG
        pltpu.async_copy(
            table_hbm.at[ids_v.at[h, j]], rows_v.at[s], gat_sems[s]
        )

    def wait_gather(s):
        pltpu.make_async_copy(
            table_hbm.at[pl.ds(0, 128)], rows_v.at[s], gat_sems[s]
        ).wait()

    def issue_store(g, s):
        h = g // NBG
        j = g % NBG
        bb = wid * NBG + j
        pltpu.async_copy(
            rows_v.at[s], out_hbm.at[pl.ds(bb * 128, 128), h, :], st_sems[s]
        )

    def wait_store(s):
        pltpu.make_async_copy(
            rows_v.at[s], out_hbm.at[pl.ds(0, 128), 0, :], st_sems[s]
        ).wait()

    for s in range(K):
        issue_gather(s, s)

    def outer(gg, carry):
        for i in range(NBUF):
            g = gg * NBUF + i
            gp = g + K
            s2 = (i + K) % NBUF

            @pl.when(jnp.logical_and(gp < G, gp >= NBUF))
            def _():
                wait_store(s2)

            @pl.when(gp < G)
            def _():
                issue_gather(gp, s2)

            wait_gather(i)
            issue_store(g, i)
        return carry

    lax.fori_loop(0, G // NBUF, outer, 0)

    for s in range(NBUF):
        wait_store(s)


_mesh = plsc.VectorSubcoreMesh(
    core_axis_name="c", subcore_axis_name="s", num_cores=NC, num_subcores=NS
)

_emb = functools.partial(
    pl.kernel,
    out_type=jax.ShapeDtypeStruct((BATCH, HIST, D), jnp.float32),
    mesh=_mesh,
    scratch_types=[
        pltpu.VMEM((HIST, NBG, 128), jnp.int32),
        pltpu.VMEM((NBUF, 128, D), jnp.float32),
    ]
    + [pltpu.SemaphoreType.DMA] * (2 * NBUF),
    compiler_params=pltpu.CompilerParams(use_tc_tiling_on_sc=False),
)(_emb_body)


def kernel(entity_ids, table):
    ids3 = entity_ids.astype(jnp.int32).T.reshape(HIST, BATCH // 128, 128)
    return _emb(ids3, table)
